# Initial kernel scaffold; baseline (speedup 1.0000x reference)
import functools

import jax
import jax.numpy as jnp
from jax import lax
from jax.experimental import pallas as pl
from jax.experimental.pallas import tpu as pltpu
from jax.experimental.pallas import tpu_sc as plsc

NUM_CORES = 2
NUM_SUBCORES = 16
NUM_TILES = NUM_CORES * NUM_SUBCORES
CHUNK = 128


def _sc_aggregate(x, src, dst, erel, rel_emb, zero_d, zero_16, onehot):
    n, d = x.shape
    e = src.shape[0]
    r = rel_emb.shape[0]
    num_chunks = e // CHUNK
    assert num_chunks * CHUNK == e
    rows_per_tile = n // NUM_SUBCORES
    assert rows_per_tile * NUM_SUBCORES == n
    full_iters = num_chunks // NUM_TILES
    tail = num_chunks - full_iters * NUM_TILES

    mesh = plsc.VectorSubcoreMesh(
        core_axis_name="c",
        subcore_axis_name="s",
        num_cores=NUM_CORES,
        num_subcores=NUM_SUBCORES,
    )

    @functools.partial(
        pl.kernel,
        out_type=(
            jax.ShapeDtypeStruct((NUM_CORES, n, d), jnp.float32),
            jax.ShapeDtypeStruct((NUM_CORES, n, 16), jnp.float32),
        ),
        mesh=mesh,
        scratch_types=[
            pltpu.VMEM_SHARED((n, d), jnp.float32),
            pltpu.VMEM_SHARED((n, 16), jnp.float32),
            pltpu.VMEM_SHARED((r, d), jnp.float32),
            pltpu.VMEM((CHUNK,), jnp.int32),
            pltpu.VMEM((CHUNK,), jnp.int32),
            pltpu.VMEM((CHUNK,), jnp.int32),
            pltpu.VMEM((CHUNK, d), jnp.float32),
            pltpu.VMEM((CHUNK, d), jnp.float32),
            pltpu.VMEM((CHUNK, 16), jnp.float32),
            pltpu.SemaphoreType.DMA,
            pltpu.SemaphoreType.DMA,
        ],
    )
    def sc_kernel(
        x_hbm, src_hbm, dst_hbm, erel_hbm, rel_hbm, zd_hbm, z16_hbm, oh_hbm,
        out_hbm, outdeg_hbm,
        accum, degacc, rel_sh,
        srcidx, dstidx, relidx, xbuf, relbuf, ohbuf,
        sem1, sem2,
    ):
        cid = lax.axis_index("c")
        sid = lax.axis_index("s")
        wid = cid * NUM_SUBCORES + sid
        row0 = sid * rows_per_tile

        @pl.when(sid == 0)
        def _():
            pltpu.sync_copy(rel_hbm, rel_sh)

        pltpu.sync_copy(zd_hbm, accum.at[pl.ds(row0, rows_per_tile), :])
        pltpu.sync_copy(z16_hbm, degacc.at[pl.ds(row0, rows_per_tile), :])
        pltpu.sync_copy(oh_hbm, ohbuf)
        plsc.subcore_barrier()

        def do_chunk(c):
            base = c * CHUNK
            pltpu.sync_copy(src_hbm.at[pl.ds(base, CHUNK)], srcidx)
            pltpu.sync_copy(erel_hbm.at[pl.ds(base, CHUNK)], relidx)
            pltpu.sync_copy(dst_hbm.at[pl.ds(base, CHUNK)], dstidx)
            cp1 = pltpu.async_copy(x_hbm.at[srcidx], xbuf, sem1)
            cp2 = pltpu.async_copy(rel_sh.at[relidx], relbuf, sem2)
            cp1.wait()
            pltpu.sync_copy(xbuf, accum.at[dstidx], add=True)
            cp2.wait()
            pltpu.sync_copy(relbuf, accum.at[dstidx], add=True)
            pltpu.sync_copy(ohbuf, degacc.at[dstidx], add=True)

        @pl.loop(0, full_iters)
        def _(i):
            do_chunk(wid + i * NUM_TILES)

        if tail:
            @pl.when(wid < tail)
            def _():
                do_chunk(full_iters * NUM_TILES + wid)

        plsc.subcore_barrier()
        pltpu.sync_copy(
            accum.at[pl.ds(row0, rows_per_tile), :],
            out_hbm.at[cid, pl.ds(row0, rows_per_tile), :],
        )
        pltpu.sync_copy(
            degacc.at[pl.ds(row0, rows_per_tile), :],
            outdeg_hbm.at[cid, pl.ds(row0, rows_per_tile), :],
        )

    return sc_kernel(x, src, dst, erel, rel_emb, zero_d, zero_16, onehot)


_SLOPE = (1.0 / 8.0 + 1.0 / 3.0) / 2.0


def _tc_combine_body(p0, p1, d0, d1, xb, wn, lw, elw, o):
    acc = p0[...] + p1[...]
    deg = d0[:, 0] + d1[:, 0]
    prec = lax.Precision.HIGHEST
    h = lax.dot(acc, wn[...], precision=prec)
    norm = 1.0 / jnp.maximum(deg, 1.0)
    loop_main = lax.dot(xb[...], lw[...], precision=prec)
    loop_evolve = lax.dot(xb[...], elw[...], precision=prec)
    loop_msg = jnp.where((deg > 0.0)[:, None], loop_main, loop_evolve)
    y = h * norm[:, None] + loop_msg
    o[...] = jnp.where(y >= 0.0, y, y * _SLOPE)


def _tc_combine(parts, degparts, x, wn, lw, elw):
    n, d = x.shape
    blk = 1000
    grid = n // blk
    assert grid * blk == n
    row_spec = pl.BlockSpec((blk, d), lambda i: (i, 0))
    deg_spec = pl.BlockSpec((blk, 16), lambda i: (i, 0))
    full_spec = pl.BlockSpec((d, d), lambda i: (0, 0))
    return pl.pallas_call(
        _tc_combine_body,
        grid=(grid,),
        in_specs=[row_spec, row_spec, deg_spec, deg_spec, row_spec,
                  full_spec, full_spec, full_spec],
        out_specs=row_spec,
        out_shape=jax.ShapeDtypeStruct((n, d), jnp.float32),
    )(parts[0], parts[1], degparts[0], degparts[1], x, wn, lw, elw)


def kernel(x, edge_index, edge_rel, rel_emb, W_neighbor, loop_weight,
           evolve_loop_weight):
    n, d = x.shape
    src = edge_index[0]
    dst = edge_index[1]
    rows_per_tile = n // NUM_SUBCORES
    zero_d = jnp.zeros((rows_per_tile, d), jnp.float32)
    zero_16 = jnp.zeros((rows_per_tile, 16), jnp.float32)
    onehot = jnp.zeros((CHUNK, 16), jnp.float32).at[:, 0].set(1.0)
    parts, degparts = _sc_aggregate(
        x, src, dst, edge_rel, rel_emb, zero_d, zero_16, onehot
    )
    return _tc_combine(parts, degparts, x, W_neighbor, loop_weight,
                       evolve_loop_weight)


# speedup vs baseline: 4.7737x; 4.7737x over previous
import functools

import jax
import jax.numpy as jnp
from jax import lax
from jax.experimental import pallas as pl
from jax.experimental.pallas import tpu as pltpu
from jax.experimental.pallas import tpu_sc as plsc

NUM_CORES = 2
NUM_SUBCORES = 16
NUM_TILES = NUM_CORES * NUM_SUBCORES
CHUNK = 80
REL_PAD = 480


def _sc_aggregate(x, src, dst, erel, rel_emb, zero_d, onehot):
    n, d = x.shape
    e = src.shape[0]
    num_chunks = e // CHUNK
    assert num_chunks * CHUNK == e
    chunks_per_tile = num_chunks // NUM_TILES
    assert chunks_per_tile * NUM_TILES == num_chunks
    rows_per_tile = -(-n // (NUM_SUBCORES * CHUNK)) * CHUNK
    n_pad = rows_per_tile * NUM_SUBCORES
    stages = rows_per_tile // CHUNK

    mesh = plsc.VectorSubcoreMesh(
        core_axis_name="c",
        subcore_axis_name="s",
        num_cores=NUM_CORES,
        num_subcores=NUM_SUBCORES,
    )

    @functools.partial(
        pl.kernel,
        out_type=(
            jax.ShapeDtypeStruct((NUM_CORES, n_pad, d), jnp.float32),
            jax.ShapeDtypeStruct((NUM_CORES, n_pad, d), jnp.float32),
        ),
        mesh=mesh,
        scratch_types=[
            pltpu.VMEM_SHARED((n_pad, d), jnp.float32),
            pltpu.VMEM((1, CHUNK), jnp.int32),
            pltpu.VMEM((1, CHUNK), jnp.int32),
            pltpu.VMEM((1, CHUNK), jnp.int32),
            pltpu.VMEM((CHUNK, d), jnp.float32),
            pltpu.VMEM((CHUNK, d), jnp.float32),
            pltpu.VMEM((CHUNK, d), jnp.float32),
            pltpu.SemaphoreType.DMA,
            pltpu.SemaphoreType.DMA,
        ],
    )
    def sc_kernel(
        x_hbm, src_hbm, dst_hbm, erel_hbm, rel_hbm, zd_hbm, oh_hbm,
        out_hbm, outdeg_hbm,
        accum,
        srcidx, dstidx, relidx, xbuf, relbuf, ohbuf,
        sem1, sem2,
    ):
        cid = lax.axis_index("c")
        sid = lax.axis_index("s")
        wid = cid * NUM_SUBCORES + sid
        row0 = sid * rows_per_tile

        pltpu.sync_copy(zd_hbm, xbuf)

        for j in range(stages):
            pltpu.sync_copy(xbuf, accum.at[pl.ds(row0 + j * CHUNK, CHUNK), :])

        pltpu.sync_copy(oh_hbm, ohbuf)
        plsc.subcore_barrier()

        def do_chunk(c):
            base = c * CHUNK
            pltpu.sync_copy(src_hbm.at[pl.ds(base, CHUNK)], srcidx.at[0])
            pltpu.sync_copy(erel_hbm.at[pl.ds(base, CHUNK)], relidx.at[0])
            pltpu.sync_copy(dst_hbm.at[pl.ds(base, CHUNK)], dstidx.at[0])
            cp1 = pltpu.async_copy(x_hbm.at[srcidx.at[0]], xbuf, sem1)
            cp2 = pltpu.async_copy(rel_hbm.at[relidx.at[0]], relbuf, sem2)
            cp1.wait()
            pltpu.sync_copy(xbuf, accum.at[dstidx.at[0]], add=True)
            cp2.wait()
            pltpu.sync_copy(relbuf, accum.at[dstidx.at[0]], add=True)

        @pl.loop(0, chunks_per_tile)
        def _(i):
            do_chunk(wid + i * NUM_TILES)

        plsc.subcore_barrier()

        @pl.loop(0, stages)
        def _(j):
            r = row0 + j * CHUNK
            pltpu.sync_copy(accum.at[pl.ds(r, CHUNK), :], xbuf)
            pltpu.sync_copy(xbuf, out_hbm.at[cid, pl.ds(r, CHUNK), :])

        plsc.subcore_barrier()

        @pl.loop(0, chunks_per_tile)
        def _(i):
            base = (wid + i * NUM_TILES) * CHUNK
            pltpu.sync_copy(dst_hbm.at[pl.ds(base, CHUNK)], dstidx.at[0])
            pltpu.sync_copy(ohbuf, accum.at[dstidx.at[0]], add=True)

        plsc.subcore_barrier()

        @pl.loop(0, stages)
        def _(j):
            r = row0 + j * CHUNK
            pltpu.sync_copy(accum.at[pl.ds(r, CHUNK), :], xbuf)
            pltpu.sync_copy(xbuf, outdeg_hbm.at[cid, pl.ds(r, CHUNK), :])

    return sc_kernel(x, src, dst, erel, rel_emb, zero_d, onehot)


_SLOPE = (1.0 / 8.0 + 1.0 / 3.0) / 2.0


def _tc_combine_body(p0, p1, q0, q1, xb, wn, lw, elw, o):
    acc = p0[...] + p1[...]
    aft = q0[:, 0] + q1[:, 0]
    deg = jnp.round(aft - acc[:, 0])
    prec = lax.Precision.HIGHEST
    h = lax.dot(acc, wn[...], precision=prec)
    norm = 1.0 / jnp.maximum(deg, 1.0)
    loop_main = lax.dot(xb[...], lw[...], precision=prec)
    loop_evolve = lax.dot(xb[...], elw[...], precision=prec)
    loop_msg = jnp.where((deg > 0.0)[:, None], loop_main, loop_evolve)
    y = h * norm[:, None] + loop_msg
    o[...] = jnp.where(y >= 0.0, y, y * _SLOPE)


def _tc_combine(parts, degparts, x, wn, lw, elw):
    n, d = x.shape
    blk = 1000
    grid = n // blk
    assert grid * blk == n
    row_spec = pl.BlockSpec((blk, d), lambda i: (i, 0))
    full_spec = pl.BlockSpec((d, d), lambda i: (0, 0))
    return pl.pallas_call(
        _tc_combine_body,
        grid=(grid,),
        in_specs=[row_spec, row_spec, row_spec, row_spec, row_spec,
                  full_spec, full_spec, full_spec],
        out_specs=row_spec,
        out_shape=jax.ShapeDtypeStruct((n, d), jnp.float32),
    )(parts[0], parts[1], degparts[0], degparts[1], x, wn, lw, elw)


def kernel(x, edge_index, edge_rel, rel_emb, W_neighbor, loop_weight,
           evolve_loop_weight):
    n, d = x.shape
    src = edge_index[0]
    dst = edge_index[1]
    rel_padded = jnp.zeros((REL_PAD, d), jnp.float32).at[
        : rel_emb.shape[0]
    ].set(rel_emb)
    zero_d = jnp.zeros((CHUNK, d), jnp.float32)
    onehot = zero_d.at[:, 0].set(1.0)
    parts, degparts = _sc_aggregate(
        x, src, dst, edge_rel, rel_padded, zero_d, onehot
    )
    return _tc_combine(parts, degparts, x, W_neighbor, loop_weight,
                       evolve_loop_weight)


# speedup vs baseline: 7.9569x; 1.6668x over previous
import functools

import jax
import jax.numpy as jnp
from jax import lax
from jax.experimental import pallas as pl
from jax.experimental.pallas import tpu as pltpu
from jax.experimental.pallas import tpu_sc as plsc

NUM_CORES = 2
NUM_SUBCORES = 16
NUM_TILES = NUM_CORES * NUM_SUBCORES
CHUNK = 80
REL_PAD = 480


def _sc_aggregate(x, idx3, rel_emb, zero_d, onehot, num_chunks):
    n, d = x.shape
    chunks_per_tile = num_chunks // NUM_TILES
    assert chunks_per_tile * NUM_TILES == num_chunks
    assert chunks_per_tile % 2 == 1
    pairs = (chunks_per_tile - 1) // 2
    rows_per_tile = -(-n // (NUM_SUBCORES * CHUNK)) * CHUNK
    n_pad = rows_per_tile * NUM_SUBCORES
    stages = rows_per_tile // CHUNK

    mesh = plsc.VectorSubcoreMesh(
        core_axis_name="c",
        subcore_axis_name="s",
        num_cores=NUM_CORES,
        num_subcores=NUM_SUBCORES,
    )

    @functools.partial(
        pl.kernel,
        out_type=(
            jax.ShapeDtypeStruct((NUM_CORES, n_pad, d), jnp.float32),
            jax.ShapeDtypeStruct((NUM_CORES, n_pad, d), jnp.float32),
        ),
        mesh=mesh,
        scratch_types=[
            pltpu.VMEM_SHARED((n_pad, d), jnp.float32),
            pltpu.VMEM((3, CHUNK), jnp.int32),
            pltpu.VMEM((3, CHUNK), jnp.int32),
            pltpu.VMEM((CHUNK, d), jnp.float32),
            pltpu.VMEM((CHUNK, d), jnp.float32),
            pltpu.VMEM((CHUNK, d), jnp.float32),
            pltpu.VMEM((CHUNK, d), jnp.float32),
            pltpu.SemaphoreType.DMA,
            pltpu.SemaphoreType.DMA,
            pltpu.SemaphoreType.DMA,
            pltpu.SemaphoreType.DMA,
            pltpu.SemaphoreType.DMA,
            pltpu.SemaphoreType.DMA,
        ],
    )
    def sc_kernel(
        x_hbm, idx3_hbm, rel_hbm, zd_hbm, oh_hbm,
        out_hbm, outdeg_hbm,
        accum,
        idx0, idx1, xb0, xb1, rb0, rb1,
        semi0, semi1, semg0, semg1, sems0, sems1,
    ):
        cid = lax.axis_index("c")
        sid = lax.axis_index("s")
        wid = cid * NUM_SUBCORES + sid
        row0 = sid * rows_per_tile
        idxs = (idx0, idx1)
        xbs = (xb0, xb1)
        rbs = (rb0, rb1)
        semis = (semi0, semi1)
        semgs = (semg0, semg1)

        def chunk_of(k):
            return wid + k * NUM_TILES

        def load_idx(k, b):
            return pltpu.async_copy(idx3_hbm.at[chunk_of(k)], idxs[b],
                                    semis[b])

        def start_gathers(b):
            pltpu.async_copy(x_hbm.at[idxs[b].at[0]], xbs[b], semgs[b])
            return pltpu.async_copy(rel_hbm.at[idxs[b].at[1]], rbs[b],
                                    semgs[b])

        def wait_idx(b):
            pltpu.make_async_copy(idx3_hbm.at[0], idxs[b], semis[b]).wait()

        def wait_gathers(b):
            pltpu.make_async_copy(x_hbm.at[idxs[b].at[0]], xbs[b],
                                  semgs[b]).wait()
            pltpu.make_async_copy(rel_hbm.at[idxs[b].at[1]], rbs[b],
                                  semgs[b]).wait()

        def scatter(b):
            cpx = pltpu.async_copy(xbs[b], accum.at[idxs[b].at[2]], sems0,
                                   add=True)
            cpr = pltpu.async_copy(rbs[b], accum.at[idxs[b].at[2]], sems1,
                                   add=True)
            cpx.wait()
            cpr.wait()

        pltpu.sync_copy(zd_hbm, xb0)

        for j in range(stages):
            pltpu.sync_copy(xb0, accum.at[pl.ds(row0 + j * CHUNK, CHUNK), :])

        plsc.subcore_barrier()

        load_idx(0, 0).wait()
        g0 = start_gathers(0)
        load_idx(1, 1)

        @pl.loop(0, pairs)
        def _(t):
            k = 2 * t
            wait_idx(1)
            wait_gathers(0)
            start_gathers(1)
            scatter(0)
            load_idx(k + 2, 0)
            wait_idx(0)
            wait_gathers(1)
            start_gathers(0)
            scatter(1)
            load_idx(k + 3, 1)

        wait_idx(1)
        wait_gathers(0)
        scatter(0)

        plsc.subcore_barrier()

        @pl.loop(0, stages)
        def _(j):
            r = row0 + j * CHUNK
            pltpu.sync_copy(accum.at[pl.ds(r, CHUNK), :], xb0)
            pltpu.sync_copy(xb0, out_hbm.at[cid, pl.ds(r, CHUNK), :])

        plsc.subcore_barrier()

        pltpu.sync_copy(oh_hbm, rb0)
        load_idx(0, 0).wait()
        load_idx(1, 1)

        @pl.loop(0, pairs)
        def _(t):
            k = 2 * t
            pltpu.async_copy(rb0, accum.at[idxs[0].at[2]], sems0,
                             add=True).wait()
            load_idx(k + 2, 0)
            wait_idx(1)
            pltpu.async_copy(rb0, accum.at[idxs[1].at[2]], sems1,
                             add=True).wait()
            load_idx(k + 3, 1)
            wait_idx(0)

        pltpu.async_copy(rb0, accum.at[idxs[0].at[2]], sems0,
                         add=True).wait()
        wait_idx(1)

        plsc.subcore_barrier()

        @pl.loop(0, stages)
        def _(j):
            r = row0 + j * CHUNK
            pltpu.sync_copy(accum.at[pl.ds(r, CHUNK), :], xb0)
            pltpu.sync_copy(xb0, outdeg_hbm.at[cid, pl.ds(r, CHUNK), :])

    return sc_kernel(x, idx3, rel_emb, zero_d, onehot)


_SLOPE = (1.0 / 8.0 + 1.0 / 3.0) / 2.0


def _tc_combine_body(p0, p1, q0, q1, xb, wn, lw, elw, o):
    acc = p0[...] + p1[...]
    aft = q0[:, 0] + q1[:, 0]
    deg = jnp.round(aft - acc[:, 0])
    prec = lax.Precision.HIGHEST
    h = lax.dot(acc, wn[...], precision=prec)
    norm = 1.0 / jnp.maximum(deg, 1.0)
    loop_main = lax.dot(xb[...], lw[...], precision=prec)
    loop_evolve = lax.dot(xb[...], elw[...], precision=prec)
    loop_msg = jnp.where((deg > 0.0)[:, None], loop_main, loop_evolve)
    y = h * norm[:, None] + loop_msg
    o[...] = jnp.where(y >= 0.0, y, y * _SLOPE)


def _tc_combine(parts, degparts, x, wn, lw, elw):
    n, d = x.shape
    blk = 1000
    grid = n // blk
    assert grid * blk == n
    row_spec = pl.BlockSpec((blk, d), lambda i: (i, 0))
    full_spec = pl.BlockSpec((d, d), lambda i: (0, 0))
    return pl.pallas_call(
        _tc_combine_body,
        grid=(grid,),
        in_specs=[row_spec, row_spec, row_spec, row_spec, row_spec,
                  full_spec, full_spec, full_spec],
        out_specs=row_spec,
        out_shape=jax.ShapeDtypeStruct((n, d), jnp.float32),
    )(parts[0], parts[1], degparts[0], degparts[1], x, wn, lw, elw)


def kernel(x, edge_index, edge_rel, rel_emb, W_neighbor, loop_weight,
           evolve_loop_weight):
    n, d = x.shape
    e = edge_index.shape[1]
    num_chunks = e // CHUNK
    assert num_chunks * CHUNK == e
    idx3 = jnp.stack(
        [
            edge_index[0].reshape(num_chunks, CHUNK),
            edge_rel.reshape(num_chunks, CHUNK),
            edge_index[1].reshape(num_chunks, CHUNK),
        ],
        axis=1,
    )
    idx3 = jnp.concatenate(
        [idx3, jnp.zeros((2 * NUM_TILES, 3, CHUNK), jnp.int32)], axis=0
    )
    rel_padded = jnp.zeros((REL_PAD, d), jnp.float32).at[
        : rel_emb.shape[0]
    ].set(rel_emb)
    zero_d = jnp.zeros((CHUNK, d), jnp.float32)
    onehot = zero_d.at[:, 0].set(1.0)
    parts, degparts = _sc_aggregate(
        x, idx3, rel_padded, zero_d, onehot, num_chunks
    )
    return _tc_combine(parts, degparts, x, W_neighbor, loop_weight,
                       evolve_loop_weight)
